# baseline (speedup 1.0000x reference)
import functools

import jax
import jax.numpy as jnp
from jax import lax
from jax.experimental import pallas as pl
from jax.experimental.pallas import tpu as pltpu
from jax.experimental.pallas import tpu_sc as plsc

M = 16384
E = 20
D = 64
P = 4096
ED = E * D

_NW = 32
_RPW = P // _NW
_CH = 64
_BM = 128
_NB = M // _BM
_K = 128
_PB = 1024


def _gather_body(h_hbm, k_hbm, idx_hbm, out_h, out_k, idx_v, buf, sem):
    wid = lax.axis_index("s") * 2 + lax.axis_index("c")
    base = wid * _RPW
    for c in range(_RPW // _CH):
        off = base + c * _CH
        pltpu.sync_copy(idx_hbm.at[pl.ds(off, _CH)], idx_v)
        pltpu.async_copy(h_hbm.at[idx_v], buf, sem).wait()
        pltpu.sync_copy(buf, out_h.at[pl.ds(off, _CH)])
        pltpu.async_copy(k_hbm.at[idx_v], buf, sem).wait()
        pltpu.sync_copy(buf, out_k.at[pl.ds(off, _CH)])


@functools.cache
def _sc_gather():
    return pl.kernel(
        _gather_body,
        mesh=plsc.VectorSubcoreMesh(core_axis_name="c", subcore_axis_name="s"),
        out_type=[
            jax.ShapeDtypeStruct((P, ED), jnp.float32),
            jax.ShapeDtypeStruct((P, ED), jnp.float32),
        ],
        scratch_types=[
            pltpu.VMEM((_CH,), jnp.int32),
            pltpu.VMEM((_CH, ED), jnp.float32),
            pltpu.SemaphoreType.DMA,
        ],
    )


def _permute_body(u_hbm, ord_hbm, out_u, idx_v, buf, sem):
    wid = lax.axis_index("s") * 2 + lax.axis_index("c")
    base = wid * _RPW
    for c in range(_RPW // _CH):
        off = base + c * _CH
        pltpu.sync_copy(ord_hbm.at[pl.ds(off, _CH)], idx_v)
        pltpu.async_copy(u_hbm.at[idx_v], buf, sem).wait()
        pltpu.sync_copy(buf, out_u.at[pl.ds(off, _CH)])


@functools.cache
def _sc_permute():
    return pl.kernel(
        _permute_body,
        mesh=plsc.VectorSubcoreMesh(core_axis_name="c", subcore_axis_name="s"),
        out_type=jax.ShapeDtypeStruct((P, ED), jnp.float32),
        scratch_types=[
            pltpu.VMEM((_CH,), jnp.int32),
            pltpu.VMEM((_CH, ED), jnp.float32),
            pltpu.SemaphoreType.DMA,
        ],
    )


def _prep_body(es_ref, u_ref, v_ref, w_ref, esw_ref, a_ref):
    esw_ref[...] = jnp.dot(es_ref[...], w_ref[...],
                           preferred_element_type=jnp.float32)
    a_ref[...] = u_ref[...] + v_ref[...]


_prep = pl.pallas_call(
    _prep_body,
    out_shape=[
        jax.ShapeDtypeStruct((P, D), jnp.float32),
        jax.ShapeDtypeStruct((D, D), jnp.float32),
    ],
)


def _update_body(ch_ref, ck_ref, es_ref, wt_ref, a_ref, r_ref, s_ref, st_ref,
                 out_ref):
    es_rep = jnp.dot(es_ref[...], r_ref[...],
                     preferred_element_type=jnp.float32)
    ch = ch_ref[...]
    pre = (ch + ck_ref[...]) * es_rep
    gates = jax.nn.sigmoid(jnp.dot(pre, s_ref[...],
                                   preferred_element_type=jnp.float32))
    a = a_ref[...]
    parts = [jnp.dot(ch[:, e * D:(e + 1) * D], a,
                     preferred_element_type=jnp.float32) for e in range(E)]
    h = jnp.maximum(jnp.concatenate(parts, axis=1) + wt_ref[...], 0.0)
    out_ref[...] = jnp.dot(gates, st_ref[...],
                           preferred_element_type=jnp.float32) * h


_update = pl.pallas_call(
    _update_body,
    grid=(P // _PB,),
    in_specs=[
        pl.BlockSpec((_PB, ED), lambda k: (k, 0)),
        pl.BlockSpec((_PB, ED), lambda k: (k, 0)),
        pl.BlockSpec((_PB, D), lambda k: (k, 0)),
        pl.BlockSpec((_PB, ED), lambda k: (0, 0)),
        pl.BlockSpec((D, D), lambda k: (0, 0)),
        pl.BlockSpec((D, ED), lambda k: (0, 0)),
        pl.BlockSpec((ED, E), lambda k: (0, 0)),
        pl.BlockSpec((E, ED), lambda k: (0, 0)),
    ],
    out_specs=pl.BlockSpec((_PB, ED), lambda k: (k, 0)),
    out_shape=jax.ShapeDtypeStruct((P, ED), jnp.float32),
)


def _apply_body(sidx_ref, starts_ref, wstart_ref, hid_ref, upd_ref, sv_ref,
                s_ref, st_ref, out_ref, acc_ref):
    b = pl.program_id(0)
    base = b * _BM
    ws = pl.multiple_of(wstart_ref[b], 8)
    t0 = starts_ref[b]
    t1 = starts_ref[b + 1]

    sv = sv_ref[pl.ds(ws, _K), :]
    tid = ws + lax.broadcasted_iota(jnp.int32, (_K, 1), 0)
    vld = (tid >= t0) & (tid < t1)
    dest = jnp.where(vld, sv - base, _BM)
    ohT = (jnp.broadcast_to(dest, (_K, _BM))
           == lax.broadcasted_iota(jnp.int32, (_K, _BM), 1)).astype(jnp.float32)
    upd_win = upd_ref[pl.ds(ws, _K), :]
    acc_ref[...] = hid_ref[...] + lax.dot_general(
        ohT, upd_win, dimension_numbers=(((0,), (0,)), ((), ())),
        preferred_element_type=jnp.float32)

    def add_one(t, carry):
        r = sidx_ref[t] - base
        acc_ref[pl.ds(r, 1), :] += upd_ref[pl.ds(t, 1), :]
        return carry

    lax.fori_loop(ws + _K, starts_ref[b + 1], add_one, 0)

    x = acc_ref[...]
    ss = jnp.dot(x * x, s_ref[...], preferred_element_type=jnp.float32)
    scale = lax.rsqrt(jnp.maximum(ss, 1e-12))
    y = x * jnp.dot(scale, st_ref[...], preferred_element_type=jnp.float32)
    out_ref[...] = y.reshape(_BM, E, D)


_apply = pl.pallas_call(
    _apply_body,
    grid_spec=pltpu.PrefetchScalarGridSpec(
        num_scalar_prefetch=3,
        grid=(_NB,),
        in_specs=[
            pl.BlockSpec((_BM, ED), lambda b, *_: (b, 0)),
            pl.BlockSpec((P, ED), lambda b, *_: (0, 0)),
            pl.BlockSpec((P, 1), lambda b, *_: (0, 0)),
            pl.BlockSpec((ED, E), lambda b, *_: (0, 0)),
            pl.BlockSpec((E, ED), lambda b, *_: (0, 0)),
        ],
        out_specs=pl.BlockSpec((_BM, E, D), lambda b, *_: (b, 0, 0)),
        scratch_shapes=[pltpu.VMEM((_BM, ED), jnp.float32)],
    ),
    out_shape=jax.ShapeDtypeStruct((M, E, D), jnp.float32),
)


def kernel(hiddens, keys, encoded_sents, indices, U, V, W):
    idx = indices.astype(jnp.int32)
    h2 = hiddens.reshape(M, ED)
    k2 = keys.reshape(M, ED)

    order = jnp.argsort(idx).astype(jnp.int32)
    sidx = jnp.take(idx, order)
    edges = jnp.arange(0, M + _BM, _BM, dtype=jnp.int32)
    starts = (sidx[None, :] < edges[:, None]).sum(axis=1, dtype=jnp.int32)
    wstart = (jnp.minimum(starts[:-1], P - _K) // 8) * 8
    sidx2 = sidx.reshape(P, 1)

    eyeD = jnp.eye(D, dtype=jnp.float32)
    R = jnp.tile(eyeD, (1, E))
    S = jnp.kron(jnp.eye(E, dtype=jnp.float32),
                 jnp.ones((D, 1), jnp.float32))
    ST = S.T

    esw, A = _prep(encoded_sents, U, V, W)
    wt1024 = jnp.tile(esw.reshape(-1), 5).reshape(_PB, ED)

    ch, ck = _sc_gather()(h2, k2, idx)
    upd = _update(ch, ck, encoded_sents, wt1024, A, R, S, ST)
    upd_s = _sc_permute()(upd, order)
    return _apply(sidx, starts, wstart, h2, upd_s, sidx2, S, ST)


# speedup vs baseline: 1.1789x; 1.1789x over previous
import functools

import jax
import jax.numpy as jnp
from jax import lax
from jax.experimental import pallas as pl
from jax.experimental.pallas import tpu as pltpu
from jax.experimental.pallas import tpu_sc as plsc

M = 16384
E = 20
D = 64
P = 4096
ED = E * D

_NW = 32
_RPW = P // _NW
_CH = 64
_BM = 128
_NB = M // _BM
_K = 128
_PB = 1024


def _gather_body(h_hbm, k_hbm, idx_hbm, out_h, out_k, idx_v, buf, sem):
    wid = lax.axis_index("s") * 2 + lax.axis_index("c")
    base = wid * _RPW
    for c in range(_RPW // _CH):
        off = base + c * _CH
        pltpu.sync_copy(idx_hbm.at[pl.ds(off, _CH)], idx_v)
        pltpu.async_copy(h_hbm.at[idx_v], buf, sem).wait()
        pltpu.sync_copy(buf, out_h.at[pl.ds(off, _CH)])
        pltpu.async_copy(k_hbm.at[idx_v], buf, sem).wait()
        pltpu.sync_copy(buf, out_k.at[pl.ds(off, _CH)])


@functools.cache
def _sc_gather():
    return pl.kernel(
        _gather_body,
        mesh=plsc.VectorSubcoreMesh(core_axis_name="c", subcore_axis_name="s"),
        out_type=[
            jax.ShapeDtypeStruct((P, ED), jnp.float32),
            jax.ShapeDtypeStruct((P, ED), jnp.float32),
        ],
        scratch_types=[
            pltpu.VMEM((_CH,), jnp.int32),
            pltpu.VMEM((_CH, ED), jnp.float32),
            pltpu.SemaphoreType.DMA,
        ],
    )


def _permute_body(u_hbm, ord_hbm, out_u, idx_v, buf, sem):
    wid = lax.axis_index("s") * 2 + lax.axis_index("c")
    base = wid * _RPW
    for c in range(_RPW // _CH):
        off = base + c * _CH
        pltpu.sync_copy(ord_hbm.at[pl.ds(off, _CH)], idx_v)
        pltpu.async_copy(u_hbm.at[idx_v], buf, sem).wait()
        pltpu.sync_copy(buf, out_u.at[pl.ds(off, _CH)])


@functools.cache
def _sc_permute():
    return pl.kernel(
        _permute_body,
        mesh=plsc.VectorSubcoreMesh(core_axis_name="c", subcore_axis_name="s"),
        out_type=jax.ShapeDtypeStruct((P, ED), jnp.float32),
        scratch_types=[
            pltpu.VMEM((_CH,), jnp.int32),
            pltpu.VMEM((_CH, ED), jnp.float32),
            pltpu.SemaphoreType.DMA,
        ],
    )


def _prep_body(es_ref, u_ref, v_ref, w_ref, esw_ref, a_ref):
    esw_ref[...] = jnp.dot(es_ref[...], w_ref[...],
                           preferred_element_type=jnp.float32)
    a_ref[...] = u_ref[...] + v_ref[...]


_prep = pl.pallas_call(
    _prep_body,
    out_shape=[
        jax.ShapeDtypeStruct((P, D), jnp.float32),
        jax.ShapeDtypeStruct((D, D), jnp.float32),
    ],
)


def _update_body(ch_ref, ck_ref, es_ref, wt_ref, a_ref, r_ref, s_ref, st_ref,
                 out_ref):
    es_rep = jnp.dot(es_ref[...], r_ref[...],
                     preferred_element_type=jnp.float32)
    ch = ch_ref[...]
    pre = (ch + ck_ref[...]) * es_rep
    gates = jax.nn.sigmoid(jnp.dot(pre, s_ref[...],
                                   preferred_element_type=jnp.float32))
    a = a_ref[...]
    parts = [jnp.dot(ch[:, e * D:(e + 1) * D], a,
                     preferred_element_type=jnp.float32) for e in range(E)]
    h = jnp.maximum(jnp.concatenate(parts, axis=1) + wt_ref[...], 0.0)
    out_ref[...] = jnp.dot(gates, st_ref[...],
                           preferred_element_type=jnp.float32) * h


_update = pl.pallas_call(
    _update_body,
    grid=(P // _PB,),
    in_specs=[
        pl.BlockSpec((_PB, ED), lambda k: (k, 0)),
        pl.BlockSpec((_PB, ED), lambda k: (k, 0)),
        pl.BlockSpec((_PB, D), lambda k: (k, 0)),
        pl.BlockSpec((_PB, ED), lambda k: (0, 0)),
        pl.BlockSpec((D, D), lambda k: (0, 0)),
        pl.BlockSpec((D, ED), lambda k: (0, 0)),
        pl.BlockSpec((ED, E), lambda k: (0, 0)),
        pl.BlockSpec((E, ED), lambda k: (0, 0)),
    ],
    out_specs=pl.BlockSpec((_PB, ED), lambda k: (k, 0)),
    out_shape=jax.ShapeDtypeStruct((P, ED), jnp.float32),
)


def _apply_body(sidx_ref, starts_ref, wstart_ref, hid_ref, upd_ref, sv_ref,
                s_ref, st_ref, out_ref):
    b = pl.program_id(0)
    base = b * _BM
    ws = pl.multiple_of(wstart_ref[b], 8)
    t0 = starts_ref[b]
    t1 = starts_ref[b + 1]

    sv = sv_ref[pl.ds(ws, _K), :]
    tid = ws + lax.broadcasted_iota(jnp.int32, (_K, 1), 0)
    vld = (tid >= t0) & (tid < t1)
    dest = jnp.where(vld, sv - base, _BM)
    ohT = (jnp.broadcast_to(dest, (_K, _BM))
           == lax.broadcasted_iota(jnp.int32, (_K, _BM), 1)).astype(jnp.float32)
    upd_win = upd_ref[pl.ds(ws, _K), :]
    out_ref[...] = hid_ref[...] + lax.dot_general(
        ohT, upd_win, dimension_numbers=(((0,), (0,)), ((), ())),
        preferred_element_type=jnp.float32)

    def add_one(t, carry):
        r = sidx_ref[t] - base
        out_ref[pl.ds(r, 1), :] += upd_ref[pl.ds(t, 1), :]
        return carry

    lax.fori_loop(ws + _K, starts_ref[b + 1], add_one, 0)

    x = out_ref[...]
    ss = jnp.dot(x * x, s_ref[...], preferred_element_type=jnp.float32)
    scale = lax.rsqrt(jnp.maximum(ss, 1e-12))
    out_ref[...] = x * jnp.dot(scale, st_ref[...],
                               preferred_element_type=jnp.float32)


_apply = pl.pallas_call(
    _apply_body,
    grid_spec=pltpu.PrefetchScalarGridSpec(
        num_scalar_prefetch=3,
        grid=(_NB,),
        in_specs=[
            pl.BlockSpec((_BM, ED), lambda b, *_: (b, 0)),
            pl.BlockSpec((P, ED), lambda b, *_: (0, 0)),
            pl.BlockSpec((P, 1), lambda b, *_: (0, 0)),
            pl.BlockSpec((ED, E), lambda b, *_: (0, 0)),
            pl.BlockSpec((E, ED), lambda b, *_: (0, 0)),
        ],
        out_specs=pl.BlockSpec((_BM, ED), lambda b, *_: (b, 0)),
    ),
    out_shape=jax.ShapeDtypeStruct((M, ED), jnp.float32),
)


def kernel(hiddens, keys, encoded_sents, indices, U, V, W):
    idx = indices.astype(jnp.int32)
    h2 = hiddens.reshape(M, ED)
    k2 = keys.reshape(M, ED)

    order = jnp.argsort(idx).astype(jnp.int32)
    sidx = jnp.take(idx, order)
    edges = jnp.arange(0, M + _BM, _BM, dtype=jnp.int32)
    starts = (sidx[None, :] < edges[:, None]).sum(axis=1, dtype=jnp.int32)
    wstart = (jnp.minimum(starts[:-1], P - _K) // 8) * 8
    sidx2 = sidx.reshape(P, 1)

    eyeD = jnp.eye(D, dtype=jnp.float32)
    R = jnp.tile(eyeD, (1, E))
    S = jnp.kron(jnp.eye(E, dtype=jnp.float32),
                 jnp.ones((D, 1), jnp.float32))
    ST = S.T

    esw, A = _prep(encoded_sents, U, V, W)
    wt1024 = jnp.tile(esw.reshape(-1), 5).reshape(_PB, ED)

    ch, ck = _sc_gather()(h2, k2, idx)
    upd = _update(ch, ck, encoded_sents, wt1024, A, R, S, ST)
    upd_s = _sc_permute()(upd, order)
    out = _apply(sidx, starts, wstart, h2, upd_s, sidx2, S, ST)
    return out.reshape(M, E, D)
